# baseline (speedup 1.0000x reference)
import jax
import jax.numpy as jnp
from jax import lax
from jax.experimental import pallas as pl
from jax.experimental.pallas import tpu as pltpu
from jax.experimental.pallas import tpu_sc as plsc

BATCH = 16384
EMBED_DIM = 64
TRUNK_DIM = 128

_NUM_CORES = 2
_NUM_SUBCORES = 16
_NW = _NUM_CORES * _NUM_SUBCORES
_NSPLIT = 2
_SUB_B = BATCH // _NSPLIT
_B_PER_W = _SUB_B // _NW
_CHUNK = 128
_NCHUNK = _B_PER_W // _CHUNK


def _gather_body(idx_hbm, table_hbm, out_hbm, idx_v, rows_v, sem):
  wid = lax.axis_index("s") * _NUM_CORES + lax.axis_index("c")
  base = wid * _B_PER_W
  pltpu.sync_copy(idx_hbm.at[pl.ds(wid * _NCHUNK, _NCHUNK)], idx_v)
  copies = []
  for j in range(_NCHUNK):
    copies.append(
        pltpu.async_copy(
            table_hbm.at[idx_v.at[j]],
            rows_v.at[pl.ds(j * _CHUNK, _CHUNK)],
            sem,
        )
    )
  for c in copies:
    c.wait()
  pltpu.sync_copy(rows_v, out_hbm.at[pl.ds(base, _B_PER_W)])


def _sc_gather(cls_idx, table):
  idx2d = cls_idx.reshape(_NW * _NCHUNK, _CHUNK)
  mesh = plsc.VectorSubcoreMesh(core_axis_name="c", subcore_axis_name="s")
  return pl.kernel(
      _gather_body,
      out_type=jax.ShapeDtypeStruct((_SUB_B, EMBED_DIM), jnp.float32),
      mesh=mesh,
      compiler_params=pltpu.CompilerParams(use_tc_tiling_on_sc=False),
      scratch_types=[
          pltpu.VMEM((_NCHUNK, _CHUNK), jnp.int32),
          pltpu.VMEM((_B_PER_W, EMBED_DIM), jnp.float32),
          pltpu.SemaphoreType.DMA,
      ],
  )(idx2d, table)


_BLK = 2048


def _mlp_body(emb_ref, w1_ref, b1_ref, w2_ref, b2_ref, out_ref):
  h = jnp.dot(emb_ref[...], w1_ref[...], preferred_element_type=jnp.float32)
  h = h + b1_ref[...]
  h = h * jax.nn.sigmoid(h)
  o = jnp.dot(h, w2_ref[...], preferred_element_type=jnp.float32)
  out_ref[...] = o + b2_ref[...]


def _tc_mlp(emb, W1, b1, W2, b2):
  grid = (_SUB_B // _BLK,)
  return pl.pallas_call(
      _mlp_body,
      grid=grid,
      in_specs=[
          pl.BlockSpec((_BLK, EMBED_DIM), lambda i: (i, 0)),
          pl.BlockSpec((EMBED_DIM, TRUNK_DIM), lambda i: (0, 0)),
          pl.BlockSpec((1, TRUNK_DIM), lambda i: (0, 0)),
          pl.BlockSpec((TRUNK_DIM, TRUNK_DIM), lambda i: (0, 0)),
          pl.BlockSpec((1, TRUNK_DIM), lambda i: (0, 0)),
      ],
      out_specs=pl.BlockSpec((_BLK, TRUNK_DIM), lambda i: (i, 0)),
      out_shape=jax.ShapeDtypeStruct((_SUB_B, TRUNK_DIM), jnp.float32),
  )(emb, W1, b1.reshape(1, TRUNK_DIM), W2, b2.reshape(1, TRUNK_DIM))


def kernel(cls_idx, table, W1, b1, W2, b2):
  idx = cls_idx.astype(jnp.int32)
  outs = []
  for k in range(_NSPLIT):
    emb = _sc_gather(lax.dynamic_slice(idx, (k * _SUB_B,), (_SUB_B,)), table)
    outs.append(_tc_mlp(emb, W1, b1, W2, b2))
  return jnp.concatenate(outs, axis=0)


# speedup vs baseline: 1.0174x; 1.0174x over previous
import functools

import jax
import jax.numpy as jnp
from jax import lax
from jax.experimental import pallas as pl
from jax.experimental.pallas import tpu as pltpu
from jax.experimental.pallas import tpu_sc as plsc

BATCH = 16384
EMBED_DIM = 64
TRUNK_DIM = 128

_NUM_CORES = 2
_NUM_SUBCORES = 16
_NW = _NUM_CORES * _NUM_SUBCORES
_B_PER_W = BATCH // _NW
_CHUNK = 128
_NCHUNK = _B_PER_W // _CHUNK


def _gather_body(idx_hbm, table_hbm, out_hbm, idx_v, rows_v, sem):
  wid = lax.axis_index("s") * _NUM_CORES + lax.axis_index("c")
  base = wid * _B_PER_W
  pltpu.sync_copy(idx_hbm.at[pl.ds(wid * _NCHUNK, _NCHUNK)], idx_v)
  copies = []
  for j in range(_NCHUNK):
    copies.append(
        pltpu.async_copy(
            table_hbm.at[idx_v.at[j]],
            rows_v.at[pl.ds(j * _CHUNK, _CHUNK)],
            sem,
        )
    )
  for c in copies:
    c.wait()
  pltpu.sync_copy(rows_v, out_hbm.at[pl.ds(base, _B_PER_W)])


@jax.jit
def _sc_gather(cls_idx, table):
  idx2d = cls_idx.reshape(_NW * _NCHUNK, _CHUNK)
  mesh = plsc.VectorSubcoreMesh(core_axis_name="c", subcore_axis_name="s")
  return pl.kernel(
      _gather_body,
      out_type=jax.ShapeDtypeStruct((BATCH, EMBED_DIM), jnp.float32),
      mesh=mesh,
      compiler_params=pltpu.CompilerParams(use_tc_tiling_on_sc=False),
      scratch_types=[
          pltpu.VMEM((_NCHUNK, _CHUNK), jnp.int32),
          pltpu.VMEM((_B_PER_W, EMBED_DIM), jnp.float32),
          pltpu.SemaphoreType.DMA,
      ],
  )(idx2d, table)


_BLK = 2048


def _mlp_body(emb_ref, w1_ref, b1_ref, w2_ref, b2_ref, out_ref):
  h = jnp.dot(emb_ref[...], w1_ref[...], preferred_element_type=jnp.float32)
  h = h + b1_ref[...]
  h = h * jax.nn.sigmoid(h)
  o = jnp.dot(h, w2_ref[...], preferred_element_type=jnp.float32)
  out_ref[...] = o + b2_ref[...]


@jax.jit
def _tc_mlp(emb, W1, b1, W2, b2):
  grid = (BATCH // _BLK,)
  return pl.pallas_call(
      _mlp_body,
      grid=grid,
      in_specs=[
          pl.BlockSpec((_BLK, EMBED_DIM), lambda i: (i, 0)),
          pl.BlockSpec((EMBED_DIM, TRUNK_DIM), lambda i: (0, 0)),
          pl.BlockSpec((1, TRUNK_DIM), lambda i: (0, 0)),
          pl.BlockSpec((TRUNK_DIM, TRUNK_DIM), lambda i: (0, 0)),
          pl.BlockSpec((1, TRUNK_DIM), lambda i: (0, 0)),
      ],
      out_specs=pl.BlockSpec((_BLK, TRUNK_DIM), lambda i: (i, 0)),
      out_shape=jax.ShapeDtypeStruct((BATCH, TRUNK_DIM), jnp.float32),
  )(emb, W1, b1.reshape(1, TRUNK_DIM), W2, b2.reshape(1, TRUNK_DIM))


def kernel(cls_idx, table, W1, b1, W2, b2):
  emb = _sc_gather(cls_idx.astype(jnp.int32), table)
  return _tc_mlp(emb, W1, b1, W2, b2)
